# baseline (speedup 1.0000x reference)
import functools

import jax
import jax.numpy as jnp
from jax import lax
from jax.experimental import pallas as pl
from jax.experimental.pallas import tpu as pltpu
from jax.experimental.pallas import tpu_sc as plsc

N_NODES = 10000
N_EDGES = 160000
NFEAT = 256
NHID = 256
NCLASS = 40
DW = 128
D2P = 128

NSC = 2
NTILES = 16
NPERSC = N_NODES // NSC
ACCROWS = 5120
TRASH = 5100
EPT = 10368
EPAD = EPT * NTILES
CH = 128
NBUF = 3
NTRIP = EPT // (3 * CH)
LISTN = EPT + CH
ROWS_PT = 312


def _agg_body(nh, supports, srcp, dstp, binits, outs, accs, src_s, dst_s,
              dst_ch, bufs, gsems, ssems):
    c = lax.axis_index("c")
    s = lax.axis_index("s")

    for h in range(nh):
        pltpu.sync_copy(binits[h],
                        accs[h].at[pl.ds(pl.multiple_of(s * 320, 320), 320)])

    lo = c * NPERSC

    base = pl.multiple_of(s * EPT, EPT)
    pltpu.sync_copy(srcp.at[pl.ds(base, EPT)], src_s.at[pl.ds(0, EPT)])
    pltpu.sync_copy(dstp.at[pl.ds(base, EPT)], dst_s.at[pl.ds(0, EPT)])

    def loc_body(i, _):
        dv = dst_s[pl.ds(i * 16, 16)]
        local = dv - lo
        m = (local >= 0) & (local < NPERSC)
        dst_s[pl.ds(i * 16, 16)] = jnp.where(m, local, TRASH)
        return 0

    lax.fori_loop(0, EPT // 16, loc_body, 0)

    zero16 = jnp.zeros((16,), jnp.int32)
    trash16 = jnp.full((16,), TRASH, jnp.int32)
    for j in range(CH // 16):
        src_s[pl.ds(EPT + j * 16, 16)] = zero16
        dst_s[pl.ds(EPT + j * 16, 16)] = trash16

    ntrip = NTRIP

    def fire_g(g_idx, h, b):
        pltpu.async_copy(
            supports[h].at[src_s.at[pl.ds(g_idx * CH, CH)]],
            bufs[h][b], gsems[h][b])

    def wait_g(g_idx, h, b):
        pltpu.make_async_copy(
            supports[h].at[src_s.at[pl.ds(g_idx * CH, CH)]],
            bufs[h][b], gsems[h][b]).wait()

    def fire_s(h, b):
        pltpu.async_copy(bufs[h][b], accs[h].at[dst_ch[b]], ssems[h][b],
                         add=True)

    def wait_s(h, b):
        pltpu.make_async_copy(bufs[h][b], accs[h].at[dst_ch[b]],
                              ssems[h][b]).wait()

    def stage(g_idx, b):
        for j in range(CH // 16):
            dst_ch[b][pl.ds(j * 16, 16)] = dst_s[pl.ds(g_idx * CH + j * 16,
                                                       16)]

    for h in range(nh):
        fire_g(0, h, 0)
    for j in range(3):
        for h in range(nh):
            wait_g(j, h, j)
        stage(j, j)
        for h in range(nh):
            fire_s(h, j)
        if j == 2:
            for h in range(nh):
                wait_s(h, 0)
        for h in range(nh):
            fire_g(j + 1, h, (j + 1) % 3)

    def trip_body(t, _):
        for b in range(3):
            g_idx = t * 3 + b
            for h in range(nh):
                wait_g(g_idx, h, b)
            stage(g_idx, b)
            for h in range(nh):
                fire_s(h, b)
                wait_s(h, (b + 1) % 3)
                fire_g(g_idx + 1, h, (b + 1) % 3)
        return 0

    lax.fori_loop(1, ntrip, trip_body, 0)

    for h in range(nh):
        wait_s(h, 1)
        wait_s(h, 2)
        wait_g(3 * ntrip, h, 0)

    plsc.subcore_barrier()

    out_base = c * NPERSC
    for h in range(nh):
        pltpu.sync_copy(
            accs[h].at[pl.ds(pl.multiple_of(s * ROWS_PT, 8), ROWS_PT)],
            outs[h].at[pl.ds(pl.multiple_of(out_base + s * ROWS_PT, 8),
                             ROWS_PT)])

    @pl.when(s == 0)
    def _():
        rem = NPERSC - NTILES * ROWS_PT
        for h in range(nh):
            pltpu.sync_copy(
                accs[h].at[pl.ds(NTILES * ROWS_PT, rem)],
                outs[h].at[pl.ds(pl.multiple_of(out_base + NTILES * ROWS_PT, 8),
                                 rem)])


def _flat_agg_body(nh, *refs):
    supports = refs[:nh]
    srcp, dstp = refs[nh], refs[nh + 1]
    binits = refs[nh + 2:2 * nh + 2]
    outs = refs[2 * nh + 2:3 * nh + 2]
    accs = refs[3 * nh + 2:4 * nh + 2]
    src_s, dst_s = refs[4 * nh + 2:4 * nh + 4]
    dst_ch = refs[4 * nh + 4:4 * nh + 7]
    rest = refs[4 * nh + 7:]
    bufs = tuple(rest[9 * h:9 * h + 3] for h in range(nh))
    gsems = tuple(rest[9 * h + 3:9 * h + 6] for h in range(nh))
    ssems = tuple(rest[9 * h + 6:9 * h + 9] for h in range(nh))
    _agg_body(nh, supports, srcp, dstp, binits, outs, accs, src_s, dst_s,
              dst_ch, bufs, gsems, ssems)


def _make_agg(nh):
    mesh = plsc.VectorSubcoreMesh(core_axis_name="c", subcore_axis_name="s")
    scratch = [pltpu.VMEM_SHARED((ACCROWS, DW), jnp.float32)
               for _ in range(nh)]
    scratch += [
        pltpu.VMEM((LISTN,), jnp.int32),
        pltpu.VMEM((LISTN,), jnp.int32),
        pltpu.VMEM((CH,), jnp.int32),
        pltpu.VMEM((CH,), jnp.int32),
        pltpu.VMEM((CH,), jnp.int32),
    ]
    for _ in range(nh):
        scratch += [pltpu.VMEM((CH, DW), jnp.float32) for _ in range(NBUF)]
        scratch += [pltpu.SemaphoreType.DMA for _ in range(2 * NBUF)]
    return pl.kernel(
        functools.partial(_flat_agg_body, nh),
        out_type=[jax.ShapeDtypeStruct((N_NODES, DW), jnp.float32)
                  for _ in range(nh)],
        mesh=mesh,
        scratch_types=scratch,
    )


def _mm1_body(x_ref, w_ref, o1_ref, o2_ref):
    acc = jnp.dot(x_ref[...], w_ref[...], preferred_element_type=jnp.float32)
    o1_ref[...] = acc[:, :DW]
    o2_ref[...] = acc[:, DW:]


def _relu_mm2_body(al_ref, ar_ref, w2_ref, o_ref):
    hl = jnp.maximum(al_ref[...], 0.0)
    hr = jnp.maximum(ar_ref[...], 0.0)
    o_ref[...] = (jnp.dot(hl, w2_ref[:DW], preferred_element_type=jnp.float32)
                  + jnp.dot(hr, w2_ref[DW:],
                            preferred_element_type=jnp.float32))


def _log_softmax_body(v_ref, o_ref):
    v = v_ref[...]
    mask = lax.broadcasted_iota(jnp.int32, v.shape, 1) < NCLASS
    vm = jnp.where(mask, v, -jnp.inf)
    m = jnp.max(vm, axis=1, keepdims=True)
    lse = jnp.log(jnp.sum(jnp.exp(vm - m), axis=1, keepdims=True)) + m
    o_ref[...] = v - lse


_BM = 1000


def _mm1(x, W1):
    return pl.pallas_call(
        _mm1_body,
        grid=(N_NODES // _BM,),
        in_specs=[
            pl.BlockSpec((_BM, NFEAT), lambda i: (i, 0)),
            pl.BlockSpec((NFEAT, NHID), lambda i: (0, 0)),
        ],
        out_specs=[pl.BlockSpec((_BM, DW), lambda i: (i, 0)),
                   pl.BlockSpec((_BM, DW), lambda i: (i, 0))],
        out_shape=[jax.ShapeDtypeStruct((N_NODES, DW), jnp.float32),
                   jax.ShapeDtypeStruct((N_NODES, DW), jnp.float32)],
    )(x, W1)


def _relu_mm2(aggl, aggr, W2p):
    return pl.pallas_call(
        _relu_mm2_body,
        grid=(N_NODES // _BM,),
        in_specs=[
            pl.BlockSpec((_BM, DW), lambda i: (i, 0)),
            pl.BlockSpec((_BM, DW), lambda i: (i, 0)),
            pl.BlockSpec((NHID, D2P), lambda i: (0, 0)),
        ],
        out_specs=pl.BlockSpec((_BM, D2P), lambda i: (i, 0)),
        out_shape=jax.ShapeDtypeStruct((N_NODES, D2P), jnp.float32),
    )(aggl, aggr, W2p)


def _log_softmax(v):
    return pl.pallas_call(
        _log_softmax_body,
        grid=(N_NODES // _BM,),
        in_specs=[pl.BlockSpec((_BM, D2P), lambda i: (i, 0))],
        out_specs=pl.BlockSpec((_BM, D2P), lambda i: (i, 0)),
        out_shape=jax.ShapeDtypeStruct((N_NODES, D2P), jnp.float32),
    )(v)


@jax.jit
def kernel(x, edge_index, W1, b1, W2, b2):
    src = edge_index[0].astype(jnp.int32)
    dst = edge_index[1].astype(jnp.int32)
    pad = EPAD - N_EDGES
    srcp = jnp.concatenate([src, jnp.zeros((pad,), jnp.int32)])
    dstp = jnp.concatenate([dst, jnp.full((pad,), N_NODES, jnp.int32)])

    binit1l = jnp.broadcast_to(b1[None, :DW], (320, DW))
    binit1r = jnp.broadcast_to(b1[None, DW:], (320, DW))
    b2p = jnp.pad(b2, (0, D2P - NCLASS))
    binit2 = jnp.broadcast_to(b2p[None, :], (320, D2P))
    W2p = jnp.pad(W2, ((0, 0), (0, D2P - NCLASS)))

    s1l, s1r = _mm1(x, W1)
    (agg1l,) = _make_agg(1)(s1l, srcp, dstp, binit1l)
    (agg1r,) = _make_agg(1)(s1r, srcp, dstp, binit1r)
    support2 = _relu_mm2(agg1l, agg1r, W2p)
    (agg2,) = _make_agg(1)(support2, srcp, dstp, binit2)
    out = _log_softmax(agg2)
    return out[:, :NCLASS]


# speedup vs baseline: 1.3781x; 1.3781x over previous
import functools

import jax
import jax.numpy as jnp
from jax import lax
from jax.experimental import pallas as pl
from jax.experimental.pallas import tpu as pltpu
from jax.experimental.pallas import tpu_sc as plsc

N_NODES = 10000
N_EDGES = 160000
NFEAT = 256
NHID = 256
NCLASS = 40
DW = 128
D2P = 128

NSC = 2
NTILES = 16
NPERSC = N_NODES // NSC
ACCROWS = 5120
TRASH = 5100
EPT = 10272
EPAD = EPT * NTILES
EPT2 = 5184
EPAD2 = EPT2 * NTILES * NSC
NTRIP2 = EPT2 // 96
LIST2 = EPT2 + 32
ACCF = 10240
TRASH2 = 10048
CH = 32
NBUF = 3
NTRIP = EPT // (3 * CH)
LISTN = EPT + 5 * CH
ROWS_PT = 312


def _agg_body(nh, supports, srcp, dstp, binits, outs, accs, src_s, dst_s,
              dst_ch, bufs, gsems, ssems):
    c = lax.axis_index("c")
    s = lax.axis_index("s")

    for h in range(nh):
        pltpu.sync_copy(binits[h],
                        accs[h].at[pl.ds(pl.multiple_of(s * 320, 320), 320)])

    lo = c * NPERSC

    base = pl.multiple_of(s * EPT, EPT)
    pltpu.sync_copy(srcp.at[pl.ds(base, EPT)], src_s.at[pl.ds(0, EPT)])
    pltpu.sync_copy(dstp.at[pl.ds(base, EPT)], dst_s.at[pl.ds(0, EPT)])

    def loc_body(i, _):
        dv = dst_s[pl.ds(i * 16, 16)]
        local = dv - lo
        m = (local >= 0) & (local < NPERSC)
        dst_s[pl.ds(i * 16, 16)] = jnp.where(m, local, TRASH)
        return 0

    lax.fori_loop(0, EPT // 16, loc_body, 0)

    zero16 = jnp.zeros((16,), jnp.int32)
    trash16 = jnp.full((16,), TRASH, jnp.int32)
    for j in range(CH // 16):
        src_s[pl.ds(EPT + j * 16, 16)] = zero16
        dst_s[pl.ds(EPT + j * 16, 16)] = trash16

    ntrip = NTRIP
    plsc.subcore_barrier()

    def fire_g(g_idx, h, b):
        pltpu.async_copy(
            supports[h].at[src_s.at[pl.ds(g_idx * CH, CH)]],
            bufs[h][b], gsems[h][b])

    def wait_g(g_idx, h, b):
        pltpu.make_async_copy(
            supports[h].at[src_s.at[pl.ds(g_idx * CH, CH)]],
            bufs[h][b], gsems[h][b]).wait()

    def fire_s(h, b):
        pltpu.async_copy(bufs[h][b], accs[h].at[dst_ch[b]], ssems[h][b],
                         add=True)

    def wait_s(h, b):
        pltpu.make_async_copy(bufs[h][b], accs[h].at[dst_ch[b]],
                              ssems[h][b]).wait()

    def stage(g_idx, b):
        for j in range(CH // 16):
            dst_ch[b][pl.ds(j * 16, 16)] = dst_s[pl.ds(g_idx * CH + j * 16,
                                                       16)]

    for h in range(nh):
        fire_g(0, h, 0)
    for j in range(3):
        for h in range(nh):
            wait_g(j, h, j)
        stage(j, j)
        for h in range(nh):
            fire_s(h, j)
        if j == 2:
            for h in range(nh):
                wait_s(h, 0)
        for h in range(nh):
            fire_g(j + 1, h, (j + 1) % 3)

    def trip_body(t, _):
        for b in range(3):
            g_idx = t * 3 + b
            for h in range(nh):
                wait_g(g_idx, h, b)
            stage(g_idx, b)
            for h in range(nh):
                fire_s(h, b)
                wait_s(h, (b + 1) % 3)
                fire_g(g_idx + 1, h, (b + 1) % 3)
        return 0

    lax.fori_loop(1, ntrip, trip_body, 0)

    for h in range(nh):
        wait_s(h, 1)
        wait_s(h, 2)
        wait_g(3 * ntrip, h, 0)

    plsc.subcore_barrier()

    out_base = c * NPERSC
    for h in range(nh):
        pltpu.sync_copy(
            accs[h].at[pl.ds(pl.multiple_of(s * ROWS_PT, 8), ROWS_PT)],
            outs[h].at[pl.ds(pl.multiple_of(out_base + s * ROWS_PT, 8),
                             ROWS_PT)])

    @pl.when(s == 0)
    def _():
        rem = NPERSC - NTILES * ROWS_PT
        for h in range(nh):
            pltpu.sync_copy(
                accs[h].at[pl.ds(NTILES * ROWS_PT, rem)],
                outs[h].at[pl.ds(pl.multiple_of(out_base + NTILES * ROWS_PT, 8),
                                 rem)])


def _flat_agg_body(nh, *refs):
    supports = refs[:nh]
    srcp, dstp = refs[nh], refs[nh + 1]
    binits = refs[nh + 2:2 * nh + 2]
    outs = refs[2 * nh + 2:3 * nh + 2]
    accs = refs[3 * nh + 2:4 * nh + 2]
    src_s, dst_s = refs[4 * nh + 2:4 * nh + 4]
    dst_ch = refs[4 * nh + 4:4 * nh + 7]
    rest = refs[4 * nh + 7:]
    bufs = tuple(rest[9 * h:9 * h + 3] for h in range(nh))
    gsems = tuple(rest[9 * h + 3:9 * h + 6] for h in range(nh))
    ssems = tuple(rest[9 * h + 6:9 * h + 9] for h in range(nh))
    _agg_body(nh, supports, srcp, dstp, binits, outs, accs, src_s, dst_s,
              dst_ch, bufs, gsems, ssems)


def _make_agg(nh):
    mesh = plsc.VectorSubcoreMesh(core_axis_name="c", subcore_axis_name="s")
    scratch = [pltpu.VMEM_SHARED((ACCROWS, DW), jnp.float32)
               for _ in range(nh)]
    scratch += [
        pltpu.VMEM((LISTN,), jnp.int32),
        pltpu.VMEM((LISTN,), jnp.int32),
        pltpu.VMEM((CH,), jnp.int32),
        pltpu.VMEM((CH,), jnp.int32),
        pltpu.VMEM((CH,), jnp.int32),
    ]
    for _ in range(nh):
        scratch += [pltpu.VMEM((CH, DW), jnp.float32) for _ in range(NBUF)]
        scratch += [pltpu.SemaphoreType.DMA for _ in range(2 * NBUF)]
    return pl.kernel(
        functools.partial(_flat_agg_body, nh),
        out_type=[jax.ShapeDtypeStruct((N_NODES, DW), jnp.float32)
                  for _ in range(nh)],
        mesh=mesh,
        scratch_types=scratch,
    )


def _agg2_body(support, srcp, dstp, binit, zinit, out0, out1, acc, src_s,
               dst_s, dc0, dc1, dc2, buf0, buf1, buf2, gs0, gs1, gs2,
               ss0, ss1, ss2):
    c = lax.axis_index("c")
    s = lax.axis_index("s")
    dst_ch = (dc0, dc1, dc2)
    bufs = (buf0, buf1, buf2)
    gsems = (gs0, gs1, gs2)
    ssems = (ss0, ss1, ss2)

    @pl.when(c == 0)
    def _():
        pltpu.sync_copy(binit, acc.at[pl.ds(pl.multiple_of(s * 640, 640), 640)])

    @pl.when(c == 1)
    def _():
        pltpu.sync_copy(zinit, acc.at[pl.ds(pl.multiple_of(s * 640, 640), 640)])

    base = pl.multiple_of(c * (EPAD2 // 2) + s * EPT2, EPT2)
    pltpu.sync_copy(srcp.at[pl.ds(base, EPT2)], src_s.at[pl.ds(0, EPT2)])
    pltpu.sync_copy(dstp.at[pl.ds(base, EPT2)], dst_s.at[pl.ds(0, EPT2)])

    zero16 = jnp.zeros((16,), jnp.int32)
    trash16 = jnp.full((16,), TRASH2, jnp.int32)
    for j in range(CH // 16):
        src_s[pl.ds(EPT2 + j * 16, 16)] = zero16
        dst_s[pl.ds(EPT2 + j * 16, 16)] = trash16

    plsc.subcore_barrier()

    def fire_g(g_idx, b):
        pltpu.async_copy(support.at[src_s.at[pl.ds(g_idx * CH, CH)]],
                         bufs[b], gsems[b])

    def wait_g(g_idx, b):
        pltpu.make_async_copy(support.at[src_s.at[pl.ds(g_idx * CH, CH)]],
                              bufs[b], gsems[b]).wait()

    def fire_s(b):
        pltpu.async_copy(bufs[b], acc.at[dst_ch[b]], ssems[b], add=True)

    def wait_s(b):
        pltpu.make_async_copy(bufs[b], acc.at[dst_ch[b]], ssems[b]).wait()

    def stage(g_idx, b):
        for j in range(CH // 16):
            dst_ch[b][pl.ds(j * 16, 16)] = dst_s[pl.ds(g_idx * CH + j * 16,
                                                       16)]

    fire_g(0, 0)
    for j in range(3):
        wait_g(j, j)
        stage(j, j)
        fire_s(j)
        if j == 2:
            wait_s(0)
        fire_g(j + 1, (j + 1) % 3)

    def trip_body(t, _):
        for b in range(3):
            g_idx = t * 3 + b
            wait_g(g_idx, b)
            stage(g_idx, b)
            fire_s(b)
            wait_s((b + 1) % 3)
            fire_g(g_idx + 1, (b + 1) % 3)
        return 0

    lax.fori_loop(1, NTRIP2, trip_body, 0)
    wait_s(1)
    wait_s(2)
    wait_g(3 * NTRIP2, 0)

    plsc.subcore_barrier()

    outs = (out0, out1)
    for ci in range(2):
        @pl.when((c == ci) & (s < NTILES - 1))
        def _():
            pltpu.sync_copy(
                acc.at[pl.ds(pl.multiple_of(s * 640, 8), 640)],
                outs[ci].at[pl.ds(pl.multiple_of(s * 640, 8), 640)])

        @pl.when((c == ci) & (s == NTILES - 1))
        def _():
            pltpu.sync_copy(acc.at[pl.ds(9600, 400)],
                            outs[ci].at[pl.ds(9600, 400)])


def _make_agg2():
    mesh = plsc.VectorSubcoreMesh(core_axis_name="c", subcore_axis_name="s")
    scratch = [
        pltpu.VMEM_SHARED((ACCF, DW), jnp.float32),
        pltpu.VMEM((LIST2,), jnp.int32),
        pltpu.VMEM((LIST2,), jnp.int32),
        pltpu.VMEM((CH,), jnp.int32),
        pltpu.VMEM((CH,), jnp.int32),
        pltpu.VMEM((CH,), jnp.int32),
    ]
    scratch += [pltpu.VMEM((CH, DW), jnp.float32) for _ in range(NBUF)]
    scratch += [pltpu.SemaphoreType.DMA for _ in range(2 * NBUF)]
    return pl.kernel(
        _agg2_body,
        out_type=[jax.ShapeDtypeStruct((N_NODES, DW), jnp.float32),
                  jax.ShapeDtypeStruct((N_NODES, DW), jnp.float32)],
        mesh=mesh,
        scratch_types=scratch,
    )


def _mm1_body(x_ref, w_ref, o1_ref, o2_ref):
    acc = jnp.dot(x_ref[...], w_ref[...], preferred_element_type=jnp.float32)
    o1_ref[...] = acc[:, :DW]
    o2_ref[...] = acc[:, DW:]


def _relu_mm2_body(al_ref, ar_ref, w2_ref, o_ref):
    hl = jnp.maximum(al_ref[...], 0.0)
    hr = jnp.maximum(ar_ref[...], 0.0)
    o_ref[...] = (jnp.dot(hl, w2_ref[:DW], preferred_element_type=jnp.float32)
                  + jnp.dot(hr, w2_ref[DW:],
                            preferred_element_type=jnp.float32))


def _log_softmax_body(v0_ref, v1_ref, o_ref):
    v = v0_ref[...] + v1_ref[...]
    mask = lax.broadcasted_iota(jnp.int32, v.shape, 1) < NCLASS
    vm = jnp.where(mask, v, -jnp.inf)
    m = jnp.max(vm, axis=1, keepdims=True)
    lse = jnp.log(jnp.sum(jnp.exp(vm - m), axis=1, keepdims=True)) + m
    o_ref[...] = v - lse


_BM = 1000


def _mm1(x, W1):
    return pl.pallas_call(
        _mm1_body,
        grid=(N_NODES // _BM,),
        in_specs=[
            pl.BlockSpec((_BM, NFEAT), lambda i: (i, 0)),
            pl.BlockSpec((NFEAT, NHID), lambda i: (0, 0)),
        ],
        out_specs=[pl.BlockSpec((_BM, DW), lambda i: (i, 0)),
                   pl.BlockSpec((_BM, DW), lambda i: (i, 0))],
        out_shape=[jax.ShapeDtypeStruct((N_NODES, DW), jnp.float32),
                   jax.ShapeDtypeStruct((N_NODES, DW), jnp.float32)],
    )(x, W1)


def _relu_mm2(aggl, aggr, W2p):
    return pl.pallas_call(
        _relu_mm2_body,
        grid=(N_NODES // _BM,),
        in_specs=[
            pl.BlockSpec((_BM, DW), lambda i: (i, 0)),
            pl.BlockSpec((_BM, DW), lambda i: (i, 0)),
            pl.BlockSpec((NHID, D2P), lambda i: (0, 0)),
        ],
        out_specs=pl.BlockSpec((_BM, D2P), lambda i: (i, 0)),
        out_shape=jax.ShapeDtypeStruct((N_NODES, D2P), jnp.float32),
    )(aggl, aggr, W2p)


def _log_softmax(v0, v1):
    return pl.pallas_call(
        _log_softmax_body,
        grid=(N_NODES // _BM,),
        in_specs=[pl.BlockSpec((_BM, D2P), lambda i: (i, 0)),
                  pl.BlockSpec((_BM, D2P), lambda i: (i, 0))],
        out_specs=pl.BlockSpec((_BM, D2P), lambda i: (i, 0)),
        out_shape=jax.ShapeDtypeStruct((N_NODES, D2P), jnp.float32),
    )(v0, v1)


@jax.jit
def kernel(x, edge_index, W1, b1, W2, b2):
    src = edge_index[0].astype(jnp.int32)
    dst = edge_index[1].astype(jnp.int32)
    pad = EPAD - N_EDGES
    srcp = jnp.concatenate([src, jnp.zeros((pad,), jnp.int32)])
    dstp = jnp.concatenate([dst, jnp.full((pad,), N_NODES, jnp.int32)])
    pad2 = EPAD2 - N_EDGES
    srcp2 = jnp.concatenate([src, jnp.zeros((pad2,), jnp.int32)])
    dstp2 = jnp.concatenate([dst, jnp.full((pad2,), TRASH2, jnp.int32)])

    binit1l = jnp.broadcast_to(b1[None, :DW], (320, DW))
    binit1r = jnp.broadcast_to(b1[None, DW:], (320, DW))
    b2p = jnp.pad(b2, (0, D2P - NCLASS))
    binit2 = jnp.broadcast_to(b2p[None, :], (640, D2P))
    zinit2 = jnp.zeros((640, D2P), jnp.float32)
    W2p = jnp.pad(W2, ((0, 0), (0, D2P - NCLASS)))

    s1l, s1r = _mm1(x, W1)
    (agg1l,) = _make_agg(1)(s1l, srcp, dstp, binit1l)
    (agg1r,) = _make_agg(1)(s1r, srcp, dstp, binit1r)
    support2 = _relu_mm2(agg1l, agg1r, W2p)
    p0, p1 = _make_agg2()(support2, srcp2, dstp2, binit2, zinit2)
    out = _log_softmax(p0, p1)
    return out[:, :NCLASS]


# speedup vs baseline: 1.9966x; 1.4488x over previous
import functools

import jax
import jax.numpy as jnp
from jax import lax
from jax.experimental import pallas as pl
from jax.experimental.pallas import tpu as pltpu
from jax.experimental.pallas import tpu_sc as plsc

N_NODES = 10000
N_EDGES = 160000
NFEAT = 256
NHID = 256
NCLASS = 40
DW = 128
D2P = 128

NSC = 2
NTILES = 16
NPERSC = N_NODES // NSC
ACCROWS = 5120
TRASH = 5100
EPT = 10272
EPAD = EPT * NTILES
EPT2 = 5184
EPAD2 = EPT2 * NTILES * NSC
NTRIP2 = EPT2 // 96
LIST2 = EPT2 + 32
ACCF = 10240
TRASH2 = 10048
CH = 32
NBUF = 3
NTRIP = EPT // (3 * CH)
LISTN = EPT + 5 * CH
ROWS_PT = 312


def _agg_body(nh, supports, srcp, dstp, binits, outs, accs, src_s, dst_s,
              dst_ch, bufs, gsems, ssems):
    c = lax.axis_index("c")
    s = lax.axis_index("s")

    for h in range(nh):
        pltpu.sync_copy(binits[h],
                        accs[h].at[pl.ds(pl.multiple_of(s * 320, 320), 320)])

    lo = c * NPERSC

    base = pl.multiple_of(s * EPT, EPT)
    pltpu.sync_copy(srcp.at[pl.ds(base, EPT)], src_s.at[pl.ds(0, EPT)])
    pltpu.sync_copy(dstp.at[pl.ds(base, EPT)], dst_s.at[pl.ds(0, EPT)])

    def loc_body(i, _):
        dv = dst_s[pl.ds(i * 16, 16)]
        local = dv - lo
        m = (local >= 0) & (local < NPERSC)
        dst_s[pl.ds(i * 16, 16)] = jnp.where(m, local, TRASH)
        return 0

    lax.fori_loop(0, EPT // 16, loc_body, 0)

    zero16 = jnp.zeros((16,), jnp.int32)
    trash16 = jnp.full((16,), TRASH, jnp.int32)
    for j in range(CH // 16):
        src_s[pl.ds(EPT + j * 16, 16)] = zero16
        dst_s[pl.ds(EPT + j * 16, 16)] = trash16

    ntrip = NTRIP
    plsc.subcore_barrier()

    def fire_g(g_idx, h, b):
        pltpu.async_copy(
            supports[h].at[src_s.at[pl.ds(g_idx * CH, CH)]],
            bufs[h][b], gsems[h][b])

    def wait_g(g_idx, h, b):
        pltpu.make_async_copy(
            supports[h].at[src_s.at[pl.ds(g_idx * CH, CH)]],
            bufs[h][b], gsems[h][b]).wait()

    def fire_s(h, b):
        pltpu.async_copy(bufs[h][b], accs[h].at[dst_ch[b]], ssems[h][b],
                         add=True)

    def wait_s(h, b):
        pltpu.make_async_copy(bufs[h][b], accs[h].at[dst_ch[b]],
                              ssems[h][b]).wait()

    def stage(g_idx, b):
        for j in range(CH // 16):
            dst_ch[b][pl.ds(j * 16, 16)] = dst_s[pl.ds(g_idx * CH + j * 16,
                                                       16)]

    for h in range(nh):
        fire_g(0, h, 0)
    for j in range(3):
        for h in range(nh):
            wait_g(j, h, j)
        stage(j, j)
        for h in range(nh):
            fire_s(h, j)
        if j == 2:
            for h in range(nh):
                wait_s(h, 0)
        for h in range(nh):
            fire_g(j + 1, h, (j + 1) % 3)

    def trip_body(t, _):
        for b in range(3):
            g_idx = t * 3 + b
            for h in range(nh):
                wait_g(g_idx, h, b)
            stage(g_idx, b)
            for h in range(nh):
                fire_s(h, b)
                wait_s(h, (b + 1) % 3)
                fire_g(g_idx + 1, h, (b + 1) % 3)
        return 0

    lax.fori_loop(1, ntrip, trip_body, 0)

    for h in range(nh):
        wait_s(h, 1)
        wait_s(h, 2)
        wait_g(3 * ntrip, h, 0)

    plsc.subcore_barrier()

    out_base = c * NPERSC
    for h in range(nh):
        pltpu.sync_copy(
            accs[h].at[pl.ds(pl.multiple_of(s * ROWS_PT, 8), ROWS_PT)],
            outs[h].at[pl.ds(pl.multiple_of(out_base + s * ROWS_PT, 8),
                             ROWS_PT)])

    @pl.when(s == 0)
    def _():
        rem = NPERSC - NTILES * ROWS_PT
        for h in range(nh):
            pltpu.sync_copy(
                accs[h].at[pl.ds(NTILES * ROWS_PT, rem)],
                outs[h].at[pl.ds(pl.multiple_of(out_base + NTILES * ROWS_PT, 8),
                                 rem)])


def _flat_agg_body(nh, *refs):
    supports = refs[:nh]
    srcp, dstp = refs[nh], refs[nh + 1]
    binits = refs[nh + 2:2 * nh + 2]
    outs = refs[2 * nh + 2:3 * nh + 2]
    accs = refs[3 * nh + 2:4 * nh + 2]
    src_s, dst_s = refs[4 * nh + 2:4 * nh + 4]
    dst_ch = refs[4 * nh + 4:4 * nh + 7]
    rest = refs[4 * nh + 7:]
    bufs = tuple(rest[9 * h:9 * h + 3] for h in range(nh))
    gsems = tuple(rest[9 * h + 3:9 * h + 6] for h in range(nh))
    ssems = tuple(rest[9 * h + 6:9 * h + 9] for h in range(nh))
    _agg_body(nh, supports, srcp, dstp, binits, outs, accs, src_s, dst_s,
              dst_ch, bufs, gsems, ssems)


def _make_agg(nh):
    mesh = plsc.VectorSubcoreMesh(core_axis_name="c", subcore_axis_name="s")
    scratch = [pltpu.VMEM_SHARED((ACCROWS, DW), jnp.float32)
               for _ in range(nh)]
    scratch += [
        pltpu.VMEM((LISTN,), jnp.int32),
        pltpu.VMEM((LISTN,), jnp.int32),
        pltpu.VMEM((CH,), jnp.int32),
        pltpu.VMEM((CH,), jnp.int32),
        pltpu.VMEM((CH,), jnp.int32),
    ]
    for _ in range(nh):
        scratch += [pltpu.VMEM((CH, DW), jnp.float32) for _ in range(NBUF)]
        scratch += [pltpu.SemaphoreType.DMA for _ in range(2 * NBUF)]
    return pl.kernel(
        functools.partial(_flat_agg_body, nh),
        out_type=[jax.ShapeDtypeStruct((N_NODES, DW), jnp.float32)
                  for _ in range(nh)],
        mesh=mesh,
        scratch_types=scratch,
    )


def _agg2_body(support, srcp, dstp, binit, zinit, out0, out1, acc, src_s,
               dst_s, dc0, dc1, dc2, buf0, buf1, buf2, gs0, gs1, gs2,
               ss0, ss1, ss2):
    c = lax.axis_index("c")
    s = lax.axis_index("s")
    dst_ch = (dc0, dc1, dc2)
    bufs = (buf0, buf1, buf2)
    gsems = (gs0, gs1, gs2)
    ssems = (ss0, ss1, ss2)

    @pl.when(c == 0)
    def _():
        pltpu.sync_copy(binit, acc.at[pl.ds(pl.multiple_of(s * 640, 640), 640)])

    @pl.when(c == 1)
    def _():
        pltpu.sync_copy(zinit, acc.at[pl.ds(pl.multiple_of(s * 640, 640), 640)])

    base = pl.multiple_of(c * (EPAD2 // 2) + s * EPT2, EPT2)
    pltpu.sync_copy(srcp.at[pl.ds(base, EPT2)], src_s.at[pl.ds(0, EPT2)])
    pltpu.sync_copy(dstp.at[pl.ds(base, EPT2)], dst_s.at[pl.ds(0, EPT2)])

    zero16 = jnp.zeros((16,), jnp.int32)
    trash16 = jnp.full((16,), TRASH2, jnp.int32)
    for j in range(CH // 16):
        src_s[pl.ds(EPT2 + j * 16, 16)] = zero16
        dst_s[pl.ds(EPT2 + j * 16, 16)] = trash16

    plsc.subcore_barrier()

    def fire_g(g_idx, b):
        pltpu.async_copy(support.at[src_s.at[pl.ds(g_idx * CH, CH)]],
                         bufs[b], gsems[b])

    def wait_g(g_idx, b):
        pltpu.make_async_copy(support.at[src_s.at[pl.ds(g_idx * CH, CH)]],
                              bufs[b], gsems[b]).wait()

    def fire_s(b):
        pltpu.async_copy(bufs[b], acc.at[dst_ch[b]], ssems[b], add=True)

    def wait_s(b):
        pltpu.make_async_copy(bufs[b], acc.at[dst_ch[b]], ssems[b]).wait()

    def stage(g_idx, b):
        for j in range(CH // 16):
            dst_ch[b][pl.ds(j * 16, 16)] = dst_s[pl.ds(g_idx * CH + j * 16,
                                                       16)]

    fire_g(0, 0)
    for j in range(3):
        wait_g(j, j)
        stage(j, j)
        fire_s(j)
        if j == 2:
            wait_s(0)
        fire_g(j + 1, (j + 1) % 3)

    def trip_body(t, _):
        for b in range(3):
            g_idx = t * 3 + b
            wait_g(g_idx, b)
            stage(g_idx, b)
            fire_s(b)
            wait_s((b + 1) % 3)
            fire_g(g_idx + 1, (b + 1) % 3)
        return 0

    lax.fori_loop(1, NTRIP2, trip_body, 0)
    wait_s(1)
    wait_s(2)
    wait_g(3 * NTRIP2, 0)

    plsc.subcore_barrier()

    outs = (out0, out1)
    for ci in range(2):
        @pl.when((c == ci) & (s < NTILES - 1))
        def _():
            pltpu.sync_copy(
                acc.at[pl.ds(pl.multiple_of(s * 640, 8), 640)],
                outs[ci].at[pl.ds(pl.multiple_of(s * 640, 8), 640)])

        @pl.when((c == ci) & (s == NTILES - 1))
        def _():
            pltpu.sync_copy(acc.at[pl.ds(9600, 400)],
                            outs[ci].at[pl.ds(9600, 400)])


def _make_agg2():
    mesh = plsc.VectorSubcoreMesh(core_axis_name="c", subcore_axis_name="s")
    scratch = [
        pltpu.VMEM_SHARED((ACCF, DW), jnp.float32),
        pltpu.VMEM((LIST2,), jnp.int32),
        pltpu.VMEM((LIST2,), jnp.int32),
        pltpu.VMEM((CH,), jnp.int32),
        pltpu.VMEM((CH,), jnp.int32),
        pltpu.VMEM((CH,), jnp.int32),
    ]
    scratch += [pltpu.VMEM((CH, DW), jnp.float32) for _ in range(NBUF)]
    scratch += [pltpu.SemaphoreType.DMA for _ in range(2 * NBUF)]
    return pl.kernel(
        _agg2_body,
        out_type=[jax.ShapeDtypeStruct((N_NODES, DW), jnp.float32),
                  jax.ShapeDtypeStruct((N_NODES, DW), jnp.float32)],
        mesh=mesh,
        scratch_types=scratch,
    )


def _mm1_body(x_ref, w_ref, o1_ref, o2_ref):
    acc = jnp.dot(x_ref[...], w_ref[...], preferred_element_type=jnp.float32)
    o1_ref[...] = acc[:, :DW]
    o2_ref[...] = acc[:, DW:]


def _relu_mm2_body(al_ref, ar_ref, w2_ref, o_ref):
    hl = jnp.maximum(al_ref[...], 0.0)
    hr = jnp.maximum(ar_ref[...], 0.0)
    o_ref[...] = (jnp.dot(hl, w2_ref[:DW], preferred_element_type=jnp.float32)
                  + jnp.dot(hr, w2_ref[DW:],
                            preferred_element_type=jnp.float32))


def _log_softmax_body(v0_ref, v1_ref, o_ref):
    v = v0_ref[...] + v1_ref[...]
    mask = lax.broadcasted_iota(jnp.int32, v.shape, 1) < NCLASS
    vm = jnp.where(mask, v, -jnp.inf)
    m = jnp.max(vm, axis=1, keepdims=True)
    lse = jnp.log(jnp.sum(jnp.exp(vm - m), axis=1, keepdims=True)) + m
    o_ref[...] = v - lse


_BM = 1000


def _mm1(x, W1):
    return pl.pallas_call(
        _mm1_body,
        grid=(N_NODES // _BM,),
        in_specs=[
            pl.BlockSpec((_BM, NFEAT), lambda i: (i, 0)),
            pl.BlockSpec((NFEAT, NHID), lambda i: (0, 0)),
        ],
        out_specs=[pl.BlockSpec((_BM, DW), lambda i: (i, 0)),
                   pl.BlockSpec((_BM, DW), lambda i: (i, 0))],
        out_shape=[jax.ShapeDtypeStruct((N_NODES, DW), jnp.float32),
                   jax.ShapeDtypeStruct((N_NODES, DW), jnp.float32)],
    )(x, W1)


def _relu_mm2(aggl, aggr, W2p):
    return pl.pallas_call(
        _relu_mm2_body,
        grid=(N_NODES // _BM,),
        in_specs=[
            pl.BlockSpec((_BM, DW), lambda i: (i, 0)),
            pl.BlockSpec((_BM, DW), lambda i: (i, 0)),
            pl.BlockSpec((NHID, D2P), lambda i: (0, 0)),
        ],
        out_specs=pl.BlockSpec((_BM, D2P), lambda i: (i, 0)),
        out_shape=jax.ShapeDtypeStruct((N_NODES, D2P), jnp.float32),
    )(aggl, aggr, W2p)


def _log_softmax(v0, v1):
    return pl.pallas_call(
        _log_softmax_body,
        grid=(N_NODES // _BM,),
        in_specs=[pl.BlockSpec((_BM, D2P), lambda i: (i, 0)),
                  pl.BlockSpec((_BM, D2P), lambda i: (i, 0))],
        out_specs=pl.BlockSpec((_BM, D2P), lambda i: (i, 0)),
        out_shape=jax.ShapeDtypeStruct((N_NODES, D2P), jnp.float32),
    )(v0, v1)


@jax.jit
def kernel(x, edge_index, W1, b1, W2, b2):
    src = edge_index[0].astype(jnp.int32)
    dst = edge_index[1].astype(jnp.int32)
    pad = EPAD - N_EDGES
    srcp = jnp.concatenate([src, jnp.zeros((pad,), jnp.int32)])
    dstp = jnp.concatenate([dst, jnp.full((pad,), N_NODES, jnp.int32)])
    pad2 = EPAD2 - N_EDGES
    srcp2 = jnp.concatenate([src, jnp.zeros((pad2,), jnp.int32)])
    dstp2 = jnp.concatenate([dst, jnp.full((pad2,), TRASH2, jnp.int32)])

    binit1l = jnp.broadcast_to(b1[None, :DW], (320, DW))
    binit1r = jnp.broadcast_to(b1[None, DW:], (320, DW))
    b2p = jnp.pad(b2, (0, D2P - NCLASS))
    binit2 = jnp.broadcast_to(b2p[None, :], (640, D2P))
    zinit2 = jnp.zeros((640, D2P), jnp.float32)
    W2p = jnp.pad(W2, ((0, 0), (0, D2P - NCLASS)))

    s1l, s1r = _mm1(x, W1)
    agg1l, agg1r = _make_agg(2)(s1l, s1r, srcp, dstp, binit1l, binit1r)
    support2 = _relu_mm2(agg1l, agg1r, W2p)
    p0, p1 = _make_agg2()(support2, srcp2, dstp2, binit2, zinit2)
    out = _log_softmax(p0, p1)
    return out[:, :NCLASS]


# speedup vs baseline: 2.0795x; 1.0415x over previous
import functools

import jax
import jax.numpy as jnp
from jax import lax
from jax.experimental import pallas as pl
from jax.experimental.pallas import tpu as pltpu
from jax.experimental.pallas import tpu_sc as plsc

N_NODES = 10000
N_EDGES = 160000
NFEAT = 256
NHID = 256
NCLASS = 40
DW = 128
D2P = 128

NSC = 2
NTILES = 16
NPERSC = N_NODES // NSC
ACCROWS = 5120
TRASH = 5100
EPT = 10272
EPAD = EPT * NTILES
EPT2 = 5184
EPAD2 = EPT2 * NTILES * NSC
NTRIP2 = EPT2 // 96
LIST2 = EPT2 + 32
ACCF = 10240
TRASH2 = 10048
CH = 32
NBUF = 3
NTRIP = EPT // (3 * CH)
LISTN = EPT + 5 * CH
ROWS_PT = 312


def _agg_body(nh, supports, srcp, dstp, binits, outs, accs, src_s, dst_s,
              dst_ch, bufs, gsems, ssems):
    c = lax.axis_index("c")
    s = lax.axis_index("s")

    for h in range(nh):
        pltpu.sync_copy(binits[h],
                        accs[h].at[pl.ds(pl.multiple_of(s * 320, 320), 320)])

    lo = c * NPERSC

    base = pl.multiple_of(s * EPT, EPT)
    pltpu.sync_copy(srcp.at[pl.ds(base, EPT)], src_s.at[pl.ds(0, EPT)])
    pltpu.sync_copy(dstp.at[pl.ds(base, EPT)], dst_s.at[pl.ds(0, EPT)])

    def loc_body(i, _):
        dv = dst_s[pl.ds(i * 16, 16)]
        local = dv - lo
        m = (local >= 0) & (local < NPERSC)
        dst_s[pl.ds(i * 16, 16)] = jnp.where(m, local, TRASH)
        return 0

    lax.fori_loop(0, EPT // 16, loc_body, 0)

    zero16 = jnp.zeros((16,), jnp.int32)
    trash16 = jnp.full((16,), TRASH, jnp.int32)
    for j in range(CH // 16):
        src_s[pl.ds(EPT + j * 16, 16)] = zero16
        dst_s[pl.ds(EPT + j * 16, 16)] = trash16

    ntrip = NTRIP
    plsc.subcore_barrier()

    def fire_g(g_idx, h, b):
        pltpu.async_copy(
            supports[h].at[src_s.at[pl.ds(g_idx * CH, CH)]],
            bufs[h][b], gsems[h][b])

    def wait_g(g_idx, h, b):
        pltpu.make_async_copy(
            supports[h].at[src_s.at[pl.ds(g_idx * CH, CH)]],
            bufs[h][b], gsems[h][b]).wait()

    def fire_s(h, b):
        pltpu.async_copy(bufs[h][b], accs[h].at[dst_ch[b]], ssems[h][b],
                         add=True)

    def wait_s(h, b):
        pltpu.make_async_copy(bufs[h][b], accs[h].at[dst_ch[b]],
                              ssems[h][b]).wait()

    def stage(g_idx, b):
        for j in range(CH // 16):
            dst_ch[b][pl.ds(j * 16, 16)] = dst_s[pl.ds(g_idx * CH + j * 16,
                                                       16)]

    for h in range(nh):
        fire_g(0, h, 0)
    for j in range(3):
        for h in range(nh):
            wait_g(j, h, j)
        stage(j, j)
        for h in range(nh):
            fire_s(h, j)
        if j == 2:
            for h in range(nh):
                wait_s(h, 0)
        for h in range(nh):
            fire_g(j + 1, h, (j + 1) % 3)

    def trip_body(t, _):
        for b in range(3):
            g_idx = t * 3 + b
            for h in range(nh):
                wait_g(g_idx, h, b)
            stage(g_idx, b)
            for h in range(nh):
                fire_s(h, b)
                wait_s(h, (b + 1) % 3)
                fire_g(g_idx + 1, h, (b + 1) % 3)
        return 0

    lax.fori_loop(1, ntrip, trip_body, 0)

    for h in range(nh):
        wait_s(h, 1)
        wait_s(h, 2)
        wait_g(3 * ntrip, h, 0)

    plsc.subcore_barrier()

    out_base = c * NPERSC
    for h in range(nh):
        pltpu.sync_copy(
            accs[h].at[pl.ds(pl.multiple_of(s * ROWS_PT, 8), ROWS_PT)],
            outs[h].at[pl.ds(pl.multiple_of(out_base + s * ROWS_PT, 8),
                             ROWS_PT)])

    @pl.when(s == 0)
    def _():
        rem = NPERSC - NTILES * ROWS_PT
        for h in range(nh):
            pltpu.sync_copy(
                accs[h].at[pl.ds(NTILES * ROWS_PT, rem)],
                outs[h].at[pl.ds(pl.multiple_of(out_base + NTILES * ROWS_PT, 8),
                                 rem)])


def _flat_agg_body(nh, *refs):
    supports = refs[:nh]
    srcp, dstp = refs[nh], refs[nh + 1]
    binits = refs[nh + 2:2 * nh + 2]
    outs = refs[2 * nh + 2:3 * nh + 2]
    accs = refs[3 * nh + 2:4 * nh + 2]
    src_s, dst_s = refs[4 * nh + 2:4 * nh + 4]
    dst_ch = refs[4 * nh + 4:4 * nh + 7]
    rest = refs[4 * nh + 7:]
    bufs = tuple(rest[9 * h:9 * h + 3] for h in range(nh))
    gsems = tuple(rest[9 * h + 3:9 * h + 6] for h in range(nh))
    ssems = tuple(rest[9 * h + 6:9 * h + 9] for h in range(nh))
    _agg_body(nh, supports, srcp, dstp, binits, outs, accs, src_s, dst_s,
              dst_ch, bufs, gsems, ssems)


def _make_agg(nh):
    mesh = plsc.VectorSubcoreMesh(core_axis_name="c", subcore_axis_name="s")
    scratch = [pltpu.VMEM_SHARED((ACCROWS, DW), jnp.float32)
               for _ in range(nh)]
    scratch += [
        pltpu.VMEM((LISTN,), jnp.int32),
        pltpu.VMEM((LISTN,), jnp.int32),
        pltpu.VMEM((CH,), jnp.int32),
        pltpu.VMEM((CH,), jnp.int32),
        pltpu.VMEM((CH,), jnp.int32),
    ]
    for _ in range(nh):
        scratch += [pltpu.VMEM((CH, DW), jnp.float32) for _ in range(NBUF)]
        scratch += [pltpu.SemaphoreType.DMA for _ in range(2 * NBUF)]
    return pl.kernel(
        functools.partial(_flat_agg_body, nh),
        out_type=[jax.ShapeDtypeStruct((N_NODES, DW), jnp.float32)
                  for _ in range(nh)],
        mesh=mesh,
        scratch_types=scratch,
    )


def _agg2_body(support, srcp, dstp, binit, zinit, out0, out1, acc, src_s,
               dst_s, *ring):
    c = lax.axis_index("c")
    s = lax.axis_index("s")
    dst_ch = (ring[0:3], ring[3:6])
    bufs = (ring[6:9], ring[9:12])
    gsems = (ring[12:15], ring[15:18])
    ssems = (ring[18:21], ring[21:24])

    @pl.when(c == 0)
    def _():
        pltpu.sync_copy(binit, acc.at[pl.ds(pl.multiple_of(s * 640, 640), 640)])

    @pl.when(c == 1)
    def _():
        pltpu.sync_copy(zinit, acc.at[pl.ds(pl.multiple_of(s * 640, 640), 640)])

    base = pl.multiple_of(c * (EPAD2 // 2) + s * EPT2, EPT2)
    pltpu.sync_copy(srcp.at[pl.ds(base, EPT2)], src_s.at[pl.ds(0, EPT2)])
    pltpu.sync_copy(dstp.at[pl.ds(base, EPT2)], dst_s.at[pl.ds(0, EPT2)])

    zero16 = jnp.zeros((16,), jnp.int32)
    trash16 = jnp.full((16,), TRASH2, jnp.int32)
    for j in range(CH // 16):
        src_s[pl.ds(EPT2 + j * 16, 16)] = zero16
        dst_s[pl.ds(EPT2 + j * 16, 16)] = trash16

    plsc.subcore_barrier()

    HC = EPT2 // (2 * CH)

    def fire_g(k, g_idx, b):
        pltpu.async_copy(
            support.at[src_s.at[pl.ds((k * HC + g_idx) * CH, CH)]],
            bufs[k][b], gsems[k][b])

    def wait_g(k, g_idx, b):
        pltpu.make_async_copy(
            support.at[src_s.at[pl.ds((k * HC + g_idx) * CH, CH)]],
            bufs[k][b], gsems[k][b]).wait()

    def fire_s(k, b):
        pltpu.async_copy(bufs[k][b], acc.at[dst_ch[k][b]], ssems[k][b],
                         add=True)

    def wait_s(k, b):
        pltpu.make_async_copy(bufs[k][b], acc.at[dst_ch[k][b]],
                              ssems[k][b]).wait()

    def stage(k, g_idx, b):
        for j in range(CH // 16):
            dst_ch[k][b][pl.ds(j * 16, 16)] = dst_s[
                pl.ds((k * HC + g_idx) * CH + j * 16, 16)]

    for k in range(2):
        fire_g(k, 0, 0)
    for j in range(3):
        for k in range(2):
            wait_g(k, j, j)
            stage(k, j, j)
            fire_s(k, j)
            if j == 2:
                wait_s(k, 0)
            fire_g(k, j + 1, (j + 1) % 3)

    def trip_body(t, _):
        for b in range(3):
            g_idx = t * 3 + b
            for k in range(2):
                wait_g(k, g_idx, b)
                stage(k, g_idx, b)
                fire_s(k, b)
                wait_s(k, (b + 1) % 3)
                fire_g(k, g_idx + 1, (b + 1) % 3)
        return 0

    lax.fori_loop(1, NTRIP2 // 2, trip_body, 0)
    for k in range(2):
        wait_s(k, 1)
        wait_s(k, 2)
        wait_g(k, HC, 0)

    plsc.subcore_barrier()

    outs = (out0, out1)
    for ci in range(2):
        @pl.when((c == ci) & (s < NTILES - 1))
        def _():
            pltpu.sync_copy(
                acc.at[pl.ds(pl.multiple_of(s * 640, 8), 640)],
                outs[ci].at[pl.ds(pl.multiple_of(s * 640, 8), 640)])

        @pl.when((c == ci) & (s == NTILES - 1))
        def _():
            pltpu.sync_copy(acc.at[pl.ds(9600, 400)],
                            outs[ci].at[pl.ds(9600, 400)])


def _make_agg2():
    mesh = plsc.VectorSubcoreMesh(core_axis_name="c", subcore_axis_name="s")
    scratch = [
        pltpu.VMEM_SHARED((ACCF, DW), jnp.float32),
        pltpu.VMEM((LIST2,), jnp.int32),
        pltpu.VMEM((LIST2,), jnp.int32),
    ]
    scratch += [pltpu.VMEM((CH,), jnp.int32) for _ in range(6)]
    scratch += [pltpu.VMEM((CH, DW), jnp.float32) for _ in range(2 * NBUF)]
    scratch += [pltpu.SemaphoreType.DMA for _ in range(4 * NBUF)]
    return pl.kernel(
        _agg2_body,
        out_type=[jax.ShapeDtypeStruct((N_NODES, DW), jnp.float32),
                  jax.ShapeDtypeStruct((N_NODES, DW), jnp.float32)],
        mesh=mesh,
        scratch_types=scratch,
    )


def _mm1_body(x_ref, w_ref, o1_ref, o2_ref):
    acc = jnp.dot(x_ref[...], w_ref[...], preferred_element_type=jnp.float32)
    o1_ref[...] = acc[:, :DW]
    o2_ref[...] = acc[:, DW:]


def _relu_mm2_body(al_ref, ar_ref, w2_ref, o_ref):
    hl = jnp.maximum(al_ref[...], 0.0)
    hr = jnp.maximum(ar_ref[...], 0.0)
    o_ref[...] = (jnp.dot(hl, w2_ref[:DW], preferred_element_type=jnp.float32)
                  + jnp.dot(hr, w2_ref[DW:],
                            preferred_element_type=jnp.float32))


def _log_softmax_body(v0_ref, v1_ref, o_ref):
    v = v0_ref[...] + v1_ref[...]
    mask = lax.broadcasted_iota(jnp.int32, v.shape, 1) < NCLASS
    vm = jnp.where(mask, v, -jnp.inf)
    m = jnp.max(vm, axis=1, keepdims=True)
    lse = jnp.log(jnp.sum(jnp.exp(vm - m), axis=1, keepdims=True)) + m
    o_ref[...] = v - lse


_BM = 1000


def _mm1(x, W1):
    return pl.pallas_call(
        _mm1_body,
        grid=(N_NODES // _BM,),
        in_specs=[
            pl.BlockSpec((_BM, NFEAT), lambda i: (i, 0)),
            pl.BlockSpec((NFEAT, NHID), lambda i: (0, 0)),
        ],
        out_specs=[pl.BlockSpec((_BM, DW), lambda i: (i, 0)),
                   pl.BlockSpec((_BM, DW), lambda i: (i, 0))],
        out_shape=[jax.ShapeDtypeStruct((N_NODES, DW), jnp.float32),
                   jax.ShapeDtypeStruct((N_NODES, DW), jnp.float32)],
    )(x, W1)


def _relu_mm2(aggl, aggr, W2p):
    return pl.pallas_call(
        _relu_mm2_body,
        grid=(N_NODES // _BM,),
        in_specs=[
            pl.BlockSpec((_BM, DW), lambda i: (i, 0)),
            pl.BlockSpec((_BM, DW), lambda i: (i, 0)),
            pl.BlockSpec((NHID, D2P), lambda i: (0, 0)),
        ],
        out_specs=pl.BlockSpec((_BM, D2P), lambda i: (i, 0)),
        out_shape=jax.ShapeDtypeStruct((N_NODES, D2P), jnp.float32),
    )(aggl, aggr, W2p)


def _log_softmax(v0, v1):
    return pl.pallas_call(
        _log_softmax_body,
        grid=(N_NODES // _BM,),
        in_specs=[pl.BlockSpec((_BM, D2P), lambda i: (i, 0)),
                  pl.BlockSpec((_BM, D2P), lambda i: (i, 0))],
        out_specs=pl.BlockSpec((_BM, D2P), lambda i: (i, 0)),
        out_shape=jax.ShapeDtypeStruct((N_NODES, D2P), jnp.float32),
    )(v0, v1)


@jax.jit
def kernel(x, edge_index, W1, b1, W2, b2):
    src = edge_index[0].astype(jnp.int32)
    dst = edge_index[1].astype(jnp.int32)
    pad = EPAD - N_EDGES
    srcp = jnp.concatenate([src, jnp.zeros((pad,), jnp.int32)])
    dstp = jnp.concatenate([dst, jnp.full((pad,), N_NODES, jnp.int32)])
    pad2 = EPAD2 - N_EDGES
    srcp2 = jnp.concatenate([src, jnp.zeros((pad2,), jnp.int32)])
    dstp2 = jnp.concatenate([dst, jnp.full((pad2,), TRASH2, jnp.int32)])

    binit1l = jnp.broadcast_to(b1[None, :DW], (320, DW))
    binit1r = jnp.broadcast_to(b1[None, DW:], (320, DW))
    b2p = jnp.pad(b2, (0, D2P - NCLASS))
    binit2 = jnp.broadcast_to(b2p[None, :], (640, D2P))
    zinit2 = jnp.zeros((640, D2P), jnp.float32)
    W2p = jnp.pad(W2, ((0, 0), (0, D2P - NCLASS)))

    s1l, s1r = _mm1(x, W1)
    agg1l, agg1r = _make_agg(2)(s1l, s1r, srcp, dstp, binit1l, binit1r)
    support2 = _relu_mm2(agg1l, agg1r, W2p)
    p0, p1 = _make_agg2()(support2, srcp2, dstp2, binit2, zinit2)
    out = _log_softmax(p0, p1)
    return out[:, :NCLASS]
